# baseline (speedup 1.0000x reference)
import functools
import jax
import jax.numpy as jnp
from jax import lax
from jax.experimental import pallas as pl
from jax.experimental.pallas import tpu as pltpu
from jax.experimental.pallas import tpu_sc as plsc

S, E, C, T = 4096, 64, 128, 256
NB = S // T
NW = 32
RPW = S // NW
CH = 16
NCH = RPW // CH

_mesh = plsc.VectorSubcoreMesh(core_axis_name="c", subcore_axis_name="s")


@functools.partial(
    pl.kernel,
    out_type=jax.ShapeDtypeStruct((S, E, C), jnp.bool_),
    mesh=_mesh,
    scratch_types=[
        pltpu.VMEM((CH, E, C), jnp.bool_),
        pltpu.SemaphoreType.DMA,
    ],
)
def _sc_zero(zc_hbm, out_hbm, zbuf, sem):
    wid = lax.axis_index("s") * 2 + lax.axis_index("c")
    pltpu.sync_copy(zc_hbm, zbuf)
    base = wid * RPW
    for k in range(NCH):
        pltpu.async_copy(zbuf, out_hbm.at[pl.ds(base + k * CH, CH)], sem)
    for k in range(NCH):
        pltpu.make_async_copy(zbuf, out_hbm.at[pl.ds(base + k * CH, CH)], sem).wait()


def _tc_zero(comb_ref):
    comb_ref[...] = jnp.zeros((T, E, C), jnp.float32)


def kernel(input_tensor, wg):
    zc = jnp.zeros((CH, E, C), jnp.bool_)
    mask = _sc_zero(zc)
    comb = pl.pallas_call(
        _tc_zero,
        grid=(NB,),
        out_specs=pl.BlockSpec((T, E, C), lambda i: (i, 0, 0)),
        out_shape=jax.ShapeDtypeStruct((S, E, C), jnp.float32),
    )()
    return (jnp.float32(0.0), comb, mask)


# speedup vs baseline: 1.3762x; 1.3762x over previous
import functools

import jax
import jax.numpy as jnp
from jax.experimental import pallas as pl
from jax.experimental.pallas import tpu as pltpu

S = 4096
D = 4096
E = 64
C = 128
T = 512
NB = S // T
T2 = 256
NB2 = S // T2

NBQ = NB

_EPS = float(jnp.finfo(jnp.float32).eps)


def _gate_pass1(x_ref, w_ref, data_ref, cnts_ref, laux_ref, acc_ref):
    i = pl.program_id(0)

    x = x_ref[...]
    w = w_ref[...]
    logits = jax.lax.dot_general(x, w, (((1,), (1,)), ((), ())),
                                 preferred_element_type=jnp.float32)

    iota_e = jax.lax.broadcasted_iota(jnp.int32, (T, E), 1).astype(jnp.float32)

    m1 = jnp.max(logits, axis=1, keepdims=True)
    idx1 = jnp.min(jnp.where(logits == m1, iota_e, float(E)), axis=1,
                   keepdims=True)
    oh1 = iota_e == idx1

    logits2 = jnp.where(oh1, -jnp.inf, logits)
    m2 = jnp.max(logits2, axis=1, keepdims=True)
    idx2 = jnp.min(jnp.where(logits2 == m2, iota_e, float(E)), axis=1,
                   keepdims=True)
    oh2 = iota_e == idx2

    ex = jnp.exp(logits - m1)
    gates = ex / jnp.sum(ex, axis=1, keepdims=True)
    g1 = jnp.sum(jnp.where(oh1, gates, 0.0), axis=1, keepdims=True)
    g2 = jnp.sum(jnp.where(oh2, gates, 0.0), axis=1, keepdims=True)

    m1f = oh1.astype(jnp.float32)
    m2f = oh2.astype(jnp.float32)

    r = jax.lax.broadcasted_iota(jnp.int32, (T, T), 0)
    c = jax.lax.broadcasted_iota(jnp.int32, (T, T), 1)
    tril = (r > c).astype(jnp.float32)
    ex1 = jnp.dot(tril, m1f, preferred_element_type=jnp.float32)
    ex2 = jnp.dot(tril, m2f, preferred_element_type=jnp.float32)
    loc1 = jnp.sum(ex1 * m1f, axis=1, keepdims=True)
    loc2 = jnp.sum(ex2 * m2f, axis=1, keepdims=True)

    cnt1 = jnp.sum(m1f, axis=0, keepdims=True)
    cnt2 = jnp.sum(m2f, axis=0, keepdims=True)

    lane = jax.lax.broadcasted_iota(jnp.int32, (T, 128), 1)
    rec = (jnp.where(lane == 0, idx1, 0.0)
           + jnp.where(lane == 1, idx2, 0.0)
           + jnp.where(lane == 2, g1, 0.0)
           + jnp.where(lane == 3, g2, 0.0)
           + jnp.where(lane == 4, loc1, 0.0)
           + jnp.where(lane == 5, loc2, 0.0))
    data_ref[0] = rec

    cnts_ref[0, 0:1, 0:E] = cnt1
    cnts_ref[0, 1:2, 0:E] = cnt2

    @pl.when(i == 0)
    def _():
        acc_ref[...] = jnp.zeros_like(acc_ref)

    acc_ref[0:1, 0:E] += jnp.sum(gates, axis=0, keepdims=True)
    acc_ref[1:2, 0:E] += cnt1

    laux_ref[...] = (float(E) / (float(S) * float(S))
                     * jnp.sum(acc_ref[0:1, 0:E] * acc_ref[1:2, 0:E],
                               keepdims=True))


def _gate_pass2(data_ref, cnts_ref, comb_ref, locsel_ref):
    i = pl.program_id(0)

    d = data_ref[0]
    idx1 = d[:, 0:1]
    idx2 = d[:, 1:2]
    g1 = d[:, 2:3]
    g2 = d[:, 3:4]
    loc1l = d[:, 4:5]
    loc2l = d[:, 5:6]

    cnts = cnts_ref[...]
    cnt1_all = cnts[:, 0, 0:E]
    cnt2_all = cnts[:, 1, 0:E]
    blk = jax.lax.broadcasted_iota(jnp.int32, (NBQ, E), 0)
    before = blk < i // (T // T2)
    offs1 = jnp.sum(jnp.where(before, cnt1_all, 0.0), axis=0, keepdims=True)
    offs2 = jnp.sum(jnp.where(before, cnt2_all, 0.0), axis=0, keepdims=True)
    total1 = jnp.sum(cnt1_all, axis=0, keepdims=True)

    iota_e = jax.lax.broadcasted_iota(jnp.int32, (T2, E), 1).astype(jnp.float32)
    oh1 = iota_e == idx1
    oh2 = iota_e == idx2
    loc1 = loc1l + jnp.sum(jnp.where(oh1, offs1, 0.0), axis=1, keepdims=True)
    loc2 = loc2l + jnp.sum(jnp.where(oh2, offs2 + total1, 0.0), axis=1,
                           keepdims=True)

    keep1 = loc1 < float(C)
    keep2 = loc2 < float(C)
    w1 = jnp.where(keep1, g1, 0.0)
    w2 = jnp.where(keep2, g2, 0.0)
    den = jnp.maximum(w1 + w2, _EPS)
    w1 = w1 / den
    w2 = w2 / den

    wsel = jnp.where(oh1, w1, 0.0) + jnp.where(oh2, w2, 0.0)
    neg = jnp.float32(-1.0)
    sel1 = jnp.where(oh1 & keep1, loc1, neg)
    sel2 = jnp.where(oh2 & keep2, loc2, neg)
    lsel = jnp.maximum(sel1, sel2)
    pv = lsel + wsel * 0.5
    pvb = pv[:, :, None]
    lf = jnp.floor(pvb)
    iota_c3 = jax.lax.broadcasted_iota(jnp.int32, (T2, E, C), 2).astype(jnp.float32)
    out = jnp.where(lf == iota_c3, (pvb - lf) * 2.0, 0.0)
    comb_ref[...] = out

    locsel_ref[...] = lsel.astype(jnp.int32).astype(jnp.int8)


@functools.partial(jax.jit, static_argnames=())
def kernel(input_tensor, wg):
    data, cnts, laux = pl.pallas_call(
        _gate_pass1,
        grid=(NB,),
        in_specs=[
            pl.BlockSpec((T, D), lambda i: (i, 0)),
            pl.BlockSpec((E, D), lambda i: (0, 0)),
        ],
        out_specs=[
            pl.BlockSpec((1, T, 128), lambda i: (i, 0, 0)),
            pl.BlockSpec((1, 2, 128), lambda i: (i, 0, 0)),
            pl.BlockSpec((1, 1), lambda i: (0, 0)),
        ],
        out_shape=[
            jax.ShapeDtypeStruct((NB, T, 128), jnp.float32),
            jax.ShapeDtypeStruct((NB, 2, 128), jnp.float32),
            jax.ShapeDtypeStruct((1, 1), jnp.float32),
        ],
        scratch_shapes=[pltpu.VMEM((2, E), jnp.float32)],
    )(input_tensor, wg)

    comb, locsel = pl.pallas_call(
        _gate_pass2,
        grid=(NB2,),
        in_specs=[
            pl.BlockSpec((1, T2, 128), lambda i: (i, 0, 0)),
            pl.BlockSpec((NB, 2, 128), lambda i: (0, 0, 0)),
        ],
        out_specs=[
            pl.BlockSpec((T2, E, C), lambda i: (i, 0, 0)),
            pl.BlockSpec((T2, E), lambda i: (i, 0)),
        ],
        out_shape=[
            jax.ShapeDtypeStruct((S, E, C), jnp.float32),
            jax.ShapeDtypeStruct((S, E), jnp.int8),
        ],
    )(data.reshape(NB2, T2, 128), cnts)

    cap_iota = jax.lax.broadcasted_iota(jnp.int8, (S, E, C), 2)
    mask = locsel[:, :, None] == cap_iota

    return (laux[0, 0], comb, mask)


# speedup vs baseline: 1.3879x; 1.0085x over previous
import functools

import jax
import jax.numpy as jnp
from jax.experimental import pallas as pl
from jax.experimental.pallas import tpu as pltpu

S = 4096
D = 4096
E = 64
C = 128
T = 512
NB = S // T
T2 = 512
NB2 = S // T2

NBQ = NB

_EPS = float(jnp.finfo(jnp.float32).eps)


def _gate_pass1(x_ref, w_ref, data_ref, cnts_ref, laux_ref, acc_ref):
    i = pl.program_id(0)

    x = x_ref[...]
    w = w_ref[...]
    logits = jax.lax.dot_general(x, w, (((1,), (1,)), ((), ())),
                                 preferred_element_type=jnp.float32)

    iota_e = jax.lax.broadcasted_iota(jnp.int32, (T, E), 1).astype(jnp.float32)

    m1 = jnp.max(logits, axis=1, keepdims=True)
    idx1 = jnp.min(jnp.where(logits == m1, iota_e, float(E)), axis=1,
                   keepdims=True)
    oh1 = iota_e == idx1

    logits2 = jnp.where(oh1, -jnp.inf, logits)
    m2 = jnp.max(logits2, axis=1, keepdims=True)
    idx2 = jnp.min(jnp.where(logits2 == m2, iota_e, float(E)), axis=1,
                   keepdims=True)
    oh2 = iota_e == idx2

    ex = jnp.exp(logits - m1)
    gates = ex / jnp.sum(ex, axis=1, keepdims=True)
    g1 = jnp.sum(jnp.where(oh1, gates, 0.0), axis=1, keepdims=True)
    g2 = jnp.sum(jnp.where(oh2, gates, 0.0), axis=1, keepdims=True)

    m1f = oh1.astype(jnp.float32)
    m2f = oh2.astype(jnp.float32)

    r = jax.lax.broadcasted_iota(jnp.int32, (T, T), 0)
    c = jax.lax.broadcasted_iota(jnp.int32, (T, T), 1)
    tril = (r > c).astype(jnp.float32)
    ex1 = jnp.dot(tril, m1f, preferred_element_type=jnp.float32)
    ex2 = jnp.dot(tril, m2f, preferred_element_type=jnp.float32)
    loc1 = jnp.sum(ex1 * m1f, axis=1, keepdims=True)
    loc2 = jnp.sum(ex2 * m2f, axis=1, keepdims=True)

    cnt1 = jnp.sum(m1f, axis=0, keepdims=True)
    cnt2 = jnp.sum(m2f, axis=0, keepdims=True)

    lane = jax.lax.broadcasted_iota(jnp.int32, (T, 128), 1)
    rec = (jnp.where(lane == 0, idx1, 0.0)
           + jnp.where(lane == 1, idx2, 0.0)
           + jnp.where(lane == 2, g1, 0.0)
           + jnp.where(lane == 3, g2, 0.0)
           + jnp.where(lane == 4, loc1, 0.0)
           + jnp.where(lane == 5, loc2, 0.0))
    data_ref[0] = rec

    cnts_ref[0, 0:1, 0:E] = cnt1
    cnts_ref[0, 1:2, 0:E] = cnt2

    @pl.when(i == 0)
    def _():
        acc_ref[...] = jnp.zeros_like(acc_ref)

    acc_ref[0:1, 0:E] += jnp.sum(gates, axis=0, keepdims=True)
    acc_ref[1:2, 0:E] += cnt1

    laux_ref[...] = (float(E) / (float(S) * float(S))
                     * jnp.sum(acc_ref[0:1, 0:E] * acc_ref[1:2, 0:E],
                               keepdims=True))


def _gate_pass2(data_ref, cnts_ref, comb_ref, locsel_ref):
    i = pl.program_id(0)

    d = data_ref[0]
    idx1 = d[:, 0:1]
    idx2 = d[:, 1:2]
    g1 = d[:, 2:3]
    g2 = d[:, 3:4]
    loc1l = d[:, 4:5]
    loc2l = d[:, 5:6]

    cnts = cnts_ref[...]
    cnt1_all = cnts[:, 0, 0:E]
    cnt2_all = cnts[:, 1, 0:E]
    blk = jax.lax.broadcasted_iota(jnp.int32, (NBQ, E), 0)
    before = blk < i // (T // T2)
    offs1 = jnp.sum(jnp.where(before, cnt1_all, 0.0), axis=0, keepdims=True)
    offs2 = jnp.sum(jnp.where(before, cnt2_all, 0.0), axis=0, keepdims=True)
    total1 = jnp.sum(cnt1_all, axis=0, keepdims=True)

    iota_e = jax.lax.broadcasted_iota(jnp.int32, (T2, E), 1).astype(jnp.float32)
    oh1 = iota_e == idx1
    oh2 = iota_e == idx2
    loc1 = loc1l + jnp.sum(jnp.where(oh1, offs1, 0.0), axis=1, keepdims=True)
    loc2 = loc2l + jnp.sum(jnp.where(oh2, offs2 + total1, 0.0), axis=1,
                           keepdims=True)

    keep1 = loc1 < float(C)
    keep2 = loc2 < float(C)
    w1 = jnp.where(keep1, g1, 0.0)
    w2 = jnp.where(keep2, g2, 0.0)
    den = jnp.maximum(w1 + w2, _EPS)
    w1 = w1 / den
    w2 = w2 / den

    wsel = jnp.where(oh1, w1, 0.0) + jnp.where(oh2, w2, 0.0)
    neg = jnp.float32(-1.0)
    sel1 = jnp.where(oh1 & keep1, loc1, neg)
    sel2 = jnp.where(oh2 & keep2, loc2, neg)
    lsel = jnp.maximum(sel1, sel2)
    pv = lsel + wsel * 0.5
    pvb = pv[:, :, None]
    lf = jnp.floor(pvb)
    iota_c3 = jax.lax.broadcasted_iota(jnp.int32, (T2, E, C), 2).astype(jnp.float32)
    out = jnp.where(lf == iota_c3, (pvb - lf) * 2.0, 0.0)
    comb_ref[...] = out

    locsel_ref[...] = lsel.astype(jnp.int32).astype(jnp.int8)


@functools.partial(jax.jit, static_argnames=())
def kernel(input_tensor, wg):
    data, cnts, laux = pl.pallas_call(
        _gate_pass1,
        grid=(NB,),
        in_specs=[
            pl.BlockSpec((T, D), lambda i: (i, 0)),
            pl.BlockSpec((E, D), lambda i: (0, 0)),
        ],
        out_specs=[
            pl.BlockSpec((1, T, 128), lambda i: (i, 0, 0)),
            pl.BlockSpec((1, 2, 128), lambda i: (i, 0, 0)),
            pl.BlockSpec((1, 1), lambda i: (0, 0)),
        ],
        out_shape=[
            jax.ShapeDtypeStruct((NB, T, 128), jnp.float32),
            jax.ShapeDtypeStruct((NB, 2, 128), jnp.float32),
            jax.ShapeDtypeStruct((1, 1), jnp.float32),
        ],
        scratch_shapes=[pltpu.VMEM((2, E), jnp.float32)],
    )(input_tensor, wg)

    comb, locsel = pl.pallas_call(
        _gate_pass2,
        grid=(NB2,),
        in_specs=[
            pl.BlockSpec((1, T2, 128), lambda i: (i, 0, 0)),
            pl.BlockSpec((NB, 2, 128), lambda i: (0, 0, 0)),
        ],
        out_specs=[
            pl.BlockSpec((T2, E, C), lambda i: (i, 0, 0)),
            pl.BlockSpec((T2, E), lambda i: (i, 0)),
        ],
        out_shape=[
            jax.ShapeDtypeStruct((S, E, C), jnp.float32),
            jax.ShapeDtypeStruct((S, E), jnp.int8),
        ],
    )(data.reshape(NB2, T2, 128), cnts)

    cap_iota = jax.lax.broadcasted_iota(jnp.int8, (S, E, C), 2)
    mask = locsel[:, :, None] == cap_iota

    return (laux[0, 0], comb, mask)
